# baseline (speedup 1.0000x reference)
import functools

import jax
import jax.numpy as jnp
from jax import lax
from jax.experimental import pallas as pl
from jax.experimental.pallas import tpu as pltpu
from jax.experimental.pallas import tpu_sc as plsc

NUM_HIDDEN = 32
BATCH = 16384
NC = 2
NS = 16
NW = NC * NS
B_PER_W = BATCH // NW
L = 16


def _make_sc_kernel():
    mesh = plsc.VectorSubcoreMesh(core_axis_name="c", subcore_axis_name="s")

    @functools.partial(
        pl.kernel,
        mesh=mesh,
        compiler_params=pltpu.CompilerParams(
            needs_layout_passes=False, use_tc_tiling_on_sc=False),
        out_type=jax.ShapeDtypeStruct((BATCH,), jnp.float32),
        scratch_types=[
            pltpu.VMEM((B_PER_W,), jnp.int32),
            pltpu.VMEM((B_PER_W,), jnp.int32),
            pltpu.VMEM((B_PER_W, NUM_HIDDEN), jnp.float32),
            pltpu.VMEM((B_PER_W, NUM_HIDDEN), jnp.float32),
            pltpu.VMEM((B_PER_W,), jnp.float32),
            pltpu.SemaphoreType.DMA,
            pltpu.SemaphoreType.DMA,
        ],
    )
    def sc_kernel(uidx_hbm, iidx_hbm, user_hbm, item_hbm, out_hbm,
                  uidx_v, iidx_v, urows_v, irows_v, out_v, sem_u, sem_i):
        wid = lax.axis_index("s") * NC + lax.axis_index("c")
        base = wid * B_PER_W

        pltpu.sync_copy(uidx_hbm.at[pl.ds(base, B_PER_W)], uidx_v)
        pltpu.sync_copy(iidx_hbm.at[pl.ds(base, B_PER_W)], iidx_v)
        cp_u = pltpu.async_copy(user_hbm.at[uidx_v], urows_v, sem_u)
        cp_i = pltpu.async_copy(item_hbm.at[iidx_v], irows_v, sem_i)
        cp_u.wait()
        cp_i.wait()

        row_iota = lax.iota(jnp.int32, L)

        def group_body(g, carry):
            rows = g * L + row_iota
            acc = jnp.zeros((L,), jnp.float32)
            for h in range(NUM_HIDDEN):
                col = jnp.full((L,), h, jnp.int32)
                u = plsc.load_gather(urows_v, [rows, col])
                v = plsc.load_gather(irows_v, [rows, col])
                acc = acc + u * v
            out_v[pl.ds(g * L, L)] = acc
            return carry

        lax.fori_loop(0, B_PER_W // L, group_body, 0)

        pltpu.sync_copy(out_v, out_hbm.at[pl.ds(base, B_PER_W)])

    return sc_kernel


_SC_KERNEL = _make_sc_kernel()


@jax.jit
def kernel(indices, ratings, user_table, item_table):
    idx = indices.astype(jnp.int32)
    pred = _SC_KERNEL(idx[0], idx[1], user_table, item_table)
    return (pred, ratings)


# speedup vs baseline: 1.4767x; 1.4767x over previous
import functools

import jax
import jax.numpy as jnp
from jax import lax
from jax.experimental import pallas as pl
from jax.experimental.pallas import tpu as pltpu
from jax.experimental.pallas import tpu_sc as plsc

NUM_HIDDEN = 32
BATCH = 16384
NC = 2
NS = 16
NW = NC * NS
B_PER_W = BATCH // NW
HALF = B_PER_W // 2
L = 16


def _scalar(vec, j):
    return jnp.squeeze(lax.slice(vec, (j,), (j + 1,)))


def _make_sc_kernel():
    mesh = plsc.VectorSubcoreMesh(core_axis_name="c", subcore_axis_name="s")

    @functools.partial(
        pl.kernel,
        mesh=mesh,
        compiler_params=pltpu.CompilerParams(needs_layout_passes=False),
        out_type=jax.ShapeDtypeStruct((BATCH,), jnp.float32),
        scratch_types=[
            pltpu.VMEM((HALF,), jnp.int32),
            pltpu.VMEM((HALF,), jnp.int32),
            pltpu.VMEM((HALF, NUM_HIDDEN), jnp.float32),
            pltpu.VMEM((HALF, NUM_HIDDEN), jnp.float32),
            pltpu.VMEM((B_PER_W,), jnp.float32),
            pltpu.SemaphoreType.DMA,
            pltpu.SemaphoreType.DMA,
        ],
    )
    def sc_kernel(uidx_hbm, iidx_hbm, user_hbm, item_hbm, out_hbm,
                  uidx_v, iidx_v, urows_v, irows_v, out_v,
                  sem_u, sem_i):
        wid = lax.axis_index("s") * NC + lax.axis_index("c")
        base = wid * B_PER_W
        row_iota = lax.iota(jnp.int32, L)

        def half_body(h, carry):
            hbase = base + h * HALF
            pltpu.sync_copy(uidx_hbm.at[pl.ds(hbase, HALF)], uidx_v)
            pltpu.sync_copy(iidx_hbm.at[pl.ds(hbase, HALF)], iidx_v)

            def fetch_body(g, c):
                uvec = uidx_v[pl.ds(g * L, L)]
                ivec = iidx_v[pl.ds(g * L, L)]
                for j in range(L):
                    pltpu.async_copy(
                        user_hbm.at[pl.ds(_scalar(uvec, j), 1)],
                        urows_v.at[pl.ds(g * L + j, 1)], sem_u)
                    pltpu.async_copy(
                        item_hbm.at[pl.ds(_scalar(ivec, j), 1)],
                        irows_v.at[pl.ds(g * L + j, 1)], sem_i)
                return c

            lax.fori_loop(0, HALF // L, fetch_body, 0)
            pltpu.make_async_copy(user_hbm.at[pl.ds(0, HALF)], urows_v,
                                  sem_u).wait()
            pltpu.make_async_copy(item_hbm.at[pl.ds(0, HALF)], irows_v,
                                  sem_i).wait()

            def group_body(g, c):
                rows = g * L + row_iota
                acc = jnp.zeros((L,), jnp.float32)
                for col_h in range(NUM_HIDDEN):
                    col = jnp.full((L,), col_h, jnp.int32)
                    u = plsc.load_gather(urows_v, [rows, col])
                    v = plsc.load_gather(irows_v, [rows, col])
                    acc = acc + u * v
                out_v[pl.ds(h * HALF + g * L, L)] = acc
                return c

            lax.fori_loop(0, HALF // L, group_body, 0)
            return carry

        lax.fori_loop(0, 2, half_body, 0)
        pltpu.sync_copy(out_v, out_hbm.at[pl.ds(base, B_PER_W)])

    return sc_kernel


_SC_KERNEL = _make_sc_kernel()


@jax.jit
def kernel(indices, ratings, user_table, item_table):
    idx = indices.astype(jnp.int32)
    pred = _SC_KERNEL(idx[0], idx[1], user_table, item_table)
    return (pred, ratings)
